# baseline (speedup 1.0000x reference)
import functools

import jax
import jax.numpy as jnp
from jax import lax
from jax.experimental import pallas as pl
from jax.experimental.pallas import tpu as pltpu
from jax.experimental.pallas import tpu_sc as plsc

_VOCAB = 1000
_BATCH = 1024
_SEQ = 50
_D = _VOCAB
_N = _BATCH * _SEQ
_NW = 32
_ROWS_PER_W = _N // _NW
_K = 50
_NCHUNK = _ROWS_PER_W // _K
_BLK = 512


def _make_sc_gather():
    mesh = plsc.VectorSubcoreMesh(core_axis_name="c", subcore_axis_name="s")

    @functools.partial(
        pl.kernel,
        mesh=mesh,
        compiler_params=pltpu.CompilerParams(use_tc_tiling_on_sc=False),
        out_type=jax.ShapeDtypeStruct((_N, _D), jnp.bfloat16),
        scratch_types=[
            pltpu.VMEM((_NCHUNK, _K), jnp.int32),
            pltpu.VMEM((_K, _D), jnp.bfloat16),
            pltpu.VMEM((_K, _D), jnp.bfloat16),
            pltpu.VMEM_SHARED((_VOCAB, _D), jnp.bfloat16),
            pltpu.SemaphoreType.DMA,
            pltpu.SemaphoreType.DMA,
            pltpu.SemaphoreType.DMA,
            pltpu.SemaphoreType.DMA,
        ],
    )
    def body(table_hbm, idx_hbm, out_hbm, idx_v, rows0, rows1, tab_sp,
             g0, g1, s0, s1):
        sid = lax.axis_index("s")
        wid = sid * 2 + lax.axis_index("c")
        base = wid * _ROWS_PER_W
        pltpu.sync_copy(idx_hbm.at[wid], idx_v)

        pltpu.sync_copy(table_hbm.at[pl.ds(sid * 62, 62)],
                        tab_sp.at[pl.ds(sid * 62, 62)])

        @pl.when(sid == 0)
        def _():
            pltpu.sync_copy(table_hbm.at[pl.ds(992, 8)],
                            tab_sp.at[pl.ds(992, 8)])

        plsc.subcore_barrier()

        rows = (rows0, rows1)
        gsem = (g0, g1)
        ssem = (s0, s1)

        def gather(g, b):
            return pltpu.make_async_copy(
                tab_sp.at[idx_v.at[g]], rows[b], gsem[b])

        def store(g, b):
            return pltpu.make_async_copy(
                rows[b], out_hbm.at[pl.ds(base + g * _K, _K)], ssem[b])

        gather(0, 0).start()
        gather(0, 0).wait()
        gather(1, 1).start()
        store(0, 0).start()

        def half_step(g, b):
            gather(g, b).wait()
            store(g - 1, 1 - b).wait()
            gather(g + 1, 1 - b).start()
            store(g, b).start()

        def pair(j, carry):
            i = 2 * j + 1
            half_step(i, 1)
            half_step(i + 1, 0)
            return carry

        lax.fori_loop(0, (_NCHUNK - 2) // 2, pair, 0)

        g = _NCHUNK - 1
        gather(g, 1).wait()
        store(g - 1, 0).wait()
        store(g, 1).start()
        store(g, 1).wait()

    return body


_sc_gather = _make_sc_gather()


def _tc_upcast_body(x_ref, out_ref):
    out_ref[...] = x_ref[...].astype(jnp.float32)


def _tc_upcast(x_bf16):
    return pl.pallas_call(
        _tc_upcast_body,
        grid=(_N // _BLK,),
        in_specs=[pl.BlockSpec((_BLK, _D), lambda i: (i, 0))],
        out_specs=pl.BlockSpec((_BLK, _D), lambda i: (i, 0)),
        out_shape=jax.ShapeDtypeStruct((_N, _D), jnp.float32),
    )(x_bf16)


def kernel(inputs, table):
    idx = inputs.reshape(_NW, _NCHUNK, _K).astype(jnp.int32)
    table_bf16 = table.astype(jnp.bfloat16)
    out_bf16 = _sc_gather(table_bf16, idx)
    out = _tc_upcast(out_bf16)
    return (out.reshape(_BATCH, _SEQ, _VOCAB), None)


# speedup vs baseline: 1.6143x; 1.6143x over previous
import functools

import jax
import jax.numpy as jnp
from jax import lax
from jax.experimental import pallas as pl
from jax.experimental.pallas import tpu as pltpu
from jax.experimental.pallas import tpu_sc as plsc

_VOCAB = 1000
_BATCH = 1024
_SEQ = 50
_D = _VOCAB
_NW = 32
_ROWS_PER_W = (_BATCH * _SEQ) // _NW
_K = 32
_NCHUNK = _ROWS_PER_W // _K


def _make_gather():
    mesh = plsc.VectorSubcoreMesh(core_axis_name="c", subcore_axis_name="s")

    @functools.partial(
        pl.kernel,
        mesh=mesh,
        compiler_params=pltpu.CompilerParams(use_tc_tiling_on_sc=False),
        out_type=jax.ShapeDtypeStruct((_BATCH * _SEQ, _D), jnp.float32),
        scratch_types=[
            pltpu.VMEM((_NCHUNK, _K), jnp.int32),
            pltpu.VMEM((_K, _D), jnp.float32),
            pltpu.VMEM((_K, _D), jnp.float32),
            pltpu.VMEM_SHARED((_VOCAB, _D), jnp.float32),
            pltpu.SemaphoreType.DMA,
            pltpu.SemaphoreType.DMA,
            pltpu.SemaphoreType.DMA,
            pltpu.SemaphoreType.DMA,
        ],
    )
    def body(table_hbm, idx_hbm, out_hbm, idx_v, rows0, rows1, tab_sp,
             g0, g1, s0, s1):
        sid = lax.axis_index("s")
        wid = sid * 2 + lax.axis_index("c")
        base = wid * _ROWS_PER_W
        pltpu.sync_copy(idx_hbm.at[wid], idx_v)

        pltpu.sync_copy(table_hbm.at[pl.ds(sid * 62, 62)],
                        tab_sp.at[pl.ds(sid * 62, 62)])

        @pl.when(sid == 0)
        def _():
            pltpu.sync_copy(table_hbm.at[pl.ds(992, 8)],
                            tab_sp.at[pl.ds(992, 8)])

        plsc.subcore_barrier()

        rows = (rows0, rows1)
        gsem = (g0, g1)
        ssem = (s0, s1)

        def gather(g, b):
            return pltpu.make_async_copy(
                tab_sp.at[idx_v.at[g]], rows[b], gsem[b])

        def store(g, b):
            return pltpu.make_async_copy(
                rows[b], out_hbm.at[pl.ds(base + g * _K, _K)], ssem[b])

        gather(0, 0).start()
        gather(0, 0).wait()
        gather(1, 1).start()
        store(0, 0).start()

        def half_step(g, b):
            gather(g, b).wait()
            store(g - 1, 1 - b).wait()
            gather(g + 1, 1 - b).start()
            store(g, b).start()

        def pair(j, carry):
            i = 2 * j + 1
            half_step(i, 1)
            half_step(i + 1, 0)
            return carry

        lax.fori_loop(0, (_NCHUNK - 2) // 2, pair, 0)

        g = _NCHUNK - 1
        gather(g, 1).wait()
        store(g - 1, 0).wait()
        store(g, 1).start()
        store(g, 1).wait()

    return body


_gather_rows = _make_gather()


def kernel(inputs, table):
    idx = inputs.reshape(_NW, _NCHUNK, _K).astype(jnp.int32)
    out = _gather_rows(table, idx)
    return (out.reshape(_BATCH, _SEQ, _VOCAB), None)


# speedup vs baseline: 1.6172x; 1.0018x over previous
import functools

import jax
import jax.numpy as jnp
from jax import lax
from jax.experimental import pallas as pl
from jax.experimental.pallas import tpu as pltpu
from jax.experimental.pallas import tpu_sc as plsc

_VOCAB = 1000
_BATCH = 1024
_SEQ = 50
_D = _VOCAB
_NW = 32
_ROWS_PER_W = (_BATCH * _SEQ) // _NW
_K = 32
_NCHUNK = _ROWS_PER_W // _K


def _make_gather():
    mesh = plsc.VectorSubcoreMesh(core_axis_name="c", subcore_axis_name="s")

    @functools.partial(
        pl.kernel,
        mesh=mesh,
        compiler_params=pltpu.CompilerParams(use_tc_tiling_on_sc=False),
        out_type=jax.ShapeDtypeStruct((_BATCH * _SEQ, _D), jnp.float32),
        scratch_types=[
            pltpu.VMEM((_NCHUNK, _K), jnp.int32),
            pltpu.VMEM((_K, _D), jnp.float32),
            pltpu.VMEM((_K, _D), jnp.float32),
            pltpu.VMEM_SHARED((_VOCAB, _D), jnp.float32),
            pltpu.SemaphoreType.DMA,
            pltpu.SemaphoreType.DMA,
            pltpu.SemaphoreType.DMA,
            pltpu.SemaphoreType.DMA,
        ],
    )
    def body(table_hbm, idx_hbm, out_hbm, idx_v, rows0, rows1, tab_sp,
             g0, g1, s0, s1):
        sid = lax.axis_index("s")
        wid = sid * 2 + lax.axis_index("c")
        base = wid * _ROWS_PER_W
        pltpu.sync_copy(idx_hbm.at[wid], idx_v)

        rows = (rows0, rows1)
        gsem = (g0, g1)
        ssem = (s0, s1)

        def gather_hbm(g, b):
            return pltpu.make_async_copy(
                table_hbm.at[idx_v.at[g]], rows[b], gsem[b])

        def gather(g, b):
            return pltpu.make_async_copy(
                tab_sp.at[idx_v.at[g]], rows[b], gsem[b])

        def store(g, b):
            return pltpu.make_async_copy(
                rows[b], out_hbm.at[pl.ds(base + g * _K, _K)], ssem[b])

        gather_hbm(0, 0).start()
        gather_hbm(1, 1).start()

        pltpu.sync_copy(table_hbm.at[pl.ds(sid * 62, 62)],
                        tab_sp.at[pl.ds(sid * 62, 62)])

        @pl.when(sid == 0)
        def _():
            pltpu.sync_copy(table_hbm.at[pl.ds(992, 8)],
                            tab_sp.at[pl.ds(992, 8)])

        gather_hbm(0, 0).wait()
        store(0, 0).start()
        plsc.subcore_barrier()

        def half_step(g, b):
            gather(g, b).wait()
            store(g - 1, 1 - b).wait()
            gather(g + 1, 1 - b).start()
            store(g, b).start()

        gather_hbm(1, 1).wait()
        store(0, 0).wait()
        gather(2, 0).start()
        store(1, 1).start()

        def pair(j, carry):
            i = 2 * j + 2
            half_step(i, 0)
            half_step(i + 1, 1)
            return carry

        lax.fori_loop(0, (_NCHUNK - 4) // 2, pair, 0)

        half_step(_NCHUNK - 2, 0)

        g = _NCHUNK - 1
        gather(g, 1).wait()
        store(g - 1, 0).wait()
        store(g, 1).start()
        store(g, 1).wait()

    return body


_gather_rows = _make_gather()


def kernel(inputs, table):
    idx = inputs.reshape(_NW, _NCHUNK, _K).astype(jnp.int32)
    out = _gather_rows(table, idx)
    return (out.reshape(_BATCH, _SEQ, _VOCAB), None)
